# baseline (speedup 1.0000x reference)
import functools

import jax
import jax.numpy as jnp
from jax import lax
from jax.experimental import pallas as pl
from jax.experimental.pallas import tpu as pltpu
from jax.experimental.pallas import tpu_sc as plsc

NC = 2
NS = 16
NW = NC * NS
L = 16
S = 8
NSC = 384
TM = 2048


def _sc_body(n_per_w, n_total, m_total,
             sx, sy, sz, sxr, syr, szr, wt,
             tx, ty, tz, txr, tyr, tzr, out,
             sx_v, sy_v, sz_v, sxr_v, syr_v, szr_v, wt_v,
             tx_v, ty_v, tz_v, txr_v, tyr_v, tzr_v, t2_v, res_v):
    cid = lax.axis_index("c")
    sid = lax.axis_index("s")
    wid = sid * NC + cid
    base = wid * n_per_w
    tbase = (base // n_total) * m_total

    pltpu.sync_copy(sx.at[pl.ds(base, n_per_w)], sx_v)
    pltpu.sync_copy(sy.at[pl.ds(base, n_per_w)], sy_v)
    pltpu.sync_copy(sz.at[pl.ds(base, n_per_w)], sz_v)
    pltpu.sync_copy(sxr.at[pl.ds(base, n_per_w)], sxr_v)
    pltpu.sync_copy(syr.at[pl.ds(base, n_per_w)], syr_v)
    pltpu.sync_copy(szr.at[pl.ds(base, n_per_w)], szr_v)
    pltpu.sync_copy(wt.at[pl.ds(base, n_per_w)], wt_v)
    pltpu.sync_copy(tx.at[pl.ds(tbase, m_total)], tx_v)
    pltpu.sync_copy(ty.at[pl.ds(tbase, m_total)], ty_v)
    pltpu.sync_copy(tz.at[pl.ds(tbase, m_total)], tz_v)
    pltpu.sync_copy(txr.at[pl.ds(tbase, m_total)], txr_v)
    pltpu.sync_copy(tyr.at[pl.ds(tbase, m_total)], tyr_v)
    pltpu.sync_copy(tzr.at[pl.ds(tbase, m_total)], tzr_v)

    def prep(j, carry):
        sl = pl.ds(j * L, L)
        a = tx_v[sl]
        bb = ty_v[sl]
        c = tz_v[sl]
        t2_v[sl] = a * a + bb * bb + c * c
        return carry
    lax.fori_loop(0, m_total // L, prep, 0)

    inf = jnp.full((L,), jnp.float32(jnp.inf))

    dnums = lax.GatherDimensionNumbers(
        offset_dims=(), collapsed_slice_dims=(0,), start_index_map=(0,))

    def lane_rot(v, sh):
        perm = (lax.iota(jnp.int32, L) + sh) % L
        return lax.gather(
            v, perm[:, None], dnums, slice_sizes=(1,),
            mode=lax.GatherScatterMode.PROMISE_IN_BOUNDS)

    def lane_min(v):
        for sh in (8, 4, 2, 1):
            v = jnp.minimum(v, lane_rot(v, sh))
        return v[0]

    def chunk_body(ci, total):
        sl = pl.ds(ci * L, L)
        sxc = sx_v[sl]
        syc = sy_v[sl]
        szc = sz_v[sl]
        sxrc = sxr_v[sl]
        syrc = syr_v[sl]
        szrc = szr_v[sl]
        wtc = wt_v[sl]
        for half in range(L // S):
            bx = [jnp.full((L,), sxrc[half * S + u] * -2.0) for u in range(S)]
            by = [jnp.full((L,), syrc[half * S + u] * -2.0) for u in range(S)]
            bz = [jnp.full((L,), szrc[half * S + u] * -2.0) for u in range(S)]

            def tgt_body(ti, accs):
                accs = list(accs)
                tsl = pl.ds(ti * L, L)
                txc = txr_v[tsl]
                tyc = tyr_v[tsl]
                tzc = tzr_v[tsl]
                t2c = t2_v[tsl]
                for u in range(S):
                    v = t2c + bx[u] * txc
                    v = v + by[u] * tyc
                    v = v + bz[u] * tzc
                    accs[u] = jnp.minimum(accs[u], v)
                return tuple(accs)

            accs = lax.fori_loop(0, m_total // L, tgt_body, (inf,) * S)

            for u in range(S):
                i = half * S + u
                s2 = sxc[i] * sxc[i] + syc[i] * syc[i] + szc[i] * szc[i]
                total = total + (lane_min(accs[u]) + s2) * wtc[i]
        return total

    total = lax.fori_loop(0, n_per_w // L, chunk_body, jnp.float32(0.0))

    lane = lax.iota(jnp.int32, L)
    res_v[...] = jnp.where(lane == 0, jnp.full((L,), total), 0.0)
    pltpu.sync_copy(res_v, out.at[wid])


def _sc_chamfer(sT, sTr, tT, tTr, weights, ntc):
    B, _, N = sT.shape
    M = tT.shape[2]
    nsc = N - ntc
    n_per_w = (B * nsc) // NW
    assert B * nsc == n_per_w * NW and nsc % n_per_w == 0
    assert M % L == 0 and n_per_w % L == 0

    mesh = plsc.VectorSubcoreMesh(
        core_axis_name="c", subcore_axis_name="s",
        num_cores=NC, num_subcores=NS)
    body = functools.partial(_sc_body, n_per_w, nsc, M)
    partials = pl.kernel(
        body,
        out_type=jax.ShapeDtypeStruct((NW, L), jnp.float32),
        mesh=mesh,
        scratch_types=(
            [pltpu.VMEM((n_per_w,), jnp.float32)] * 7
            + [pltpu.VMEM((M,), jnp.float32)] * 7
            + [pltpu.VMEM((L,), jnp.float32)]
        ),
    )(
        sT[:, 0, ntc:].reshape(-1), sT[:, 1, ntc:].reshape(-1),
        sT[:, 2, ntc:].reshape(-1),
        sTr[:, 0, ntc:].reshape(-1), sTr[:, 1, ntc:].reshape(-1),
        sTr[:, 2, ntc:].reshape(-1),
        weights[:, ntc:].reshape(-1),
        tT[:, 0].reshape(-1), tT[:, 1].reshape(-1), tT[:, 2].reshape(-1),
        tTr[:, 0].reshape(-1), tTr[:, 1].reshape(-1), tTr[:, 2].reshape(-1),
    )
    return jnp.sum(partials)


def _tc_body(s_aug, t_bf, t_raw, s_raw, w, out, macc):
    mi = pl.program_id(1)
    nm = pl.num_programs(1)
    tr = t_raw[0]
    t2 = tr[0] * tr[0] + tr[1] * tr[1] + tr[2] * tr[2]
    t2_hi = t2.astype(jnp.bfloat16)
    t2_lo = (t2 - t2_hi.astype(jnp.float32)).astype(jnp.bfloat16)
    zeros = jnp.zeros((3, t2.shape[0]), jnp.bfloat16)
    t_aug = jnp.concatenate(
        [t_bf[0], t2_hi[None], t2_lo[None], zeros], axis=0)
    cand = lax.dot_general(
        s_aug[0], t_aug, (((0,), (0,)), ((), ())),
        preferred_element_type=jnp.float32)
    part = jnp.min(cand, axis=1)

    @pl.when(mi == 0)
    def _():
        macc[...] = part

    @pl.when(mi > 0)
    def _():
        macc[...] = jnp.minimum(macc[...], part)

    @pl.when(mi == nm - 1)
    def _():
        sr = s_raw[0]
        s2 = sr[0] * sr[0] + sr[1] * sr[1] + sr[2] * sr[2]
        val = jnp.sum((macc[...] + s2) * w[0, 0])
        row = lax.broadcasted_iota(jnp.int32, (8, 128), 0)
        col = lax.broadcasted_iota(jnp.int32, (8, 128), 1)
        mask = (row == 0) & (col == 0)
        out[...] = jnp.where(mask, val, 0.0)[None]


def _tc_chamfer(sT, s_aug, tT, t_bf, w3, ntc):
    B, _, N = sT.shape
    M = tT.shape[2]
    assert M % TM == 0
    out = pl.pallas_call(
        _tc_body,
        grid=(B, M // TM),
        in_specs=[
            pl.BlockSpec((1, 8, ntc), lambda b, m: (b, 0, 0)),
            pl.BlockSpec((1, 3, TM), lambda b, m: (b, 0, m)),
            pl.BlockSpec((1, 3, TM), lambda b, m: (b, 0, m)),
            pl.BlockSpec((1, 3, ntc), lambda b, m: (b, 0, 0)),
            pl.BlockSpec((1, 1, ntc), lambda b, m: (b, 0, 0)),
        ],
        out_specs=pl.BlockSpec((1, 8, 128), lambda b, m: (b, 0, 0)),
        out_shape=jax.ShapeDtypeStruct((B, 8, 128), jnp.float32),
        scratch_shapes=[pltpu.VMEM((ntc,), jnp.float32)],
    )(s_aug, t_bf, tT, sT, w3)
    return jnp.sum(out)


def kernel(source_cloud, target_cloud, weights_source):
    B, N, D = source_cloud.shape
    M = target_cloud.shape[1]
    ntc = N - NSC

    def round_bf16(x):
        u = lax.bitcast_convert_type(x, jnp.uint32)
        r = (u + jnp.uint32(0x7FFF) + ((u >> 16) & jnp.uint32(1)))
        r = r & jnp.uint32(0xFFFF0000)
        return lax.bitcast_convert_type(r, jnp.float32)

    sT = jnp.transpose(source_cloud, (0, 2, 1))
    tT = jnp.transpose(target_cloud, (0, 2, 1))
    sTr = round_bf16(sT)
    tTr = round_bf16(tT)

    s_pre = (-2.0 * sTr).astype(jnp.bfloat16)
    ones = jnp.ones((B, 2, N), jnp.bfloat16)
    zeros = jnp.zeros((B, 3, N), jnp.bfloat16)
    s_aug = jnp.concatenate([s_pre, ones, zeros], axis=1)
    t_bf = tTr.astype(jnp.bfloat16)
    w3 = weights_source[:, None, :]

    tc_sum = _tc_chamfer(sT, s_aug, tT, t_bf, w3, ntc)
    sc_sum = _sc_chamfer(sT, sTr, tT, tTr, weights_source, ntc)

    return (tc_sum + sc_sum) / B


# speedup vs baseline: 1.0111x; 1.0111x over previous
import functools

import jax
import jax.numpy as jnp
from jax import lax
from jax.experimental import pallas as pl
from jax.experimental.pallas import tpu as pltpu
from jax.experimental.pallas import tpu_sc as plsc

NC = 2
NS = 16
NW = NC * NS
L = 16
S = 8
NSC = 384
TM = 2048


def _sc_body(n_per_w, n_total, m_total,
             sx, sy, sz, sxr, syr, szr, wt,
             tx, ty, tz, txr, tyr, tzr, out,
             sx_v, sy_v, sz_v, sxr_v, syr_v, szr_v, wt_v,
             tx_v, ty_v, tz_v, txr_v, tyr_v, tzr_v, t2_v, res_v):
    cid = lax.axis_index("c")
    sid = lax.axis_index("s")
    wid = sid * NC + cid
    base = wid * n_per_w
    tbase = (base // n_total) * m_total

    pltpu.sync_copy(sx.at[pl.ds(base, n_per_w)], sx_v)
    pltpu.sync_copy(sy.at[pl.ds(base, n_per_w)], sy_v)
    pltpu.sync_copy(sz.at[pl.ds(base, n_per_w)], sz_v)
    pltpu.sync_copy(sxr.at[pl.ds(base, n_per_w)], sxr_v)
    pltpu.sync_copy(syr.at[pl.ds(base, n_per_w)], syr_v)
    pltpu.sync_copy(szr.at[pl.ds(base, n_per_w)], szr_v)
    pltpu.sync_copy(wt.at[pl.ds(base, n_per_w)], wt_v)
    pltpu.sync_copy(tx.at[pl.ds(tbase, m_total)], tx_v)
    pltpu.sync_copy(ty.at[pl.ds(tbase, m_total)], ty_v)
    pltpu.sync_copy(tz.at[pl.ds(tbase, m_total)], tz_v)
    pltpu.sync_copy(txr.at[pl.ds(tbase, m_total)], txr_v)
    pltpu.sync_copy(tyr.at[pl.ds(tbase, m_total)], tyr_v)
    pltpu.sync_copy(tzr.at[pl.ds(tbase, m_total)], tzr_v)

    def prep(j, carry):
        sl = pl.ds(j * L, L)
        a = tx_v[sl]
        bb = ty_v[sl]
        c = tz_v[sl]
        t2_v[sl] = a * a + bb * bb + c * c
        return carry
    lax.fori_loop(0, m_total // L, prep, 0)

    inf = jnp.full((L,), jnp.float32(jnp.inf))

    dnums = lax.GatherDimensionNumbers(
        offset_dims=(), collapsed_slice_dims=(0,), start_index_map=(0,))

    def lane_rot(v, sh):
        perm = (lax.iota(jnp.int32, L) + sh) % L
        return lax.gather(
            v, perm[:, None], dnums, slice_sizes=(1,),
            mode=lax.GatherScatterMode.PROMISE_IN_BOUNDS)

    def lane_min(v):
        for sh in (8, 4, 2, 1):
            v = jnp.minimum(v, lane_rot(v, sh))
        return v[0]

    def chunk_body(ci, total):
        sl = pl.ds(ci * L, L)
        sxc = sx_v[sl]
        syc = sy_v[sl]
        szc = sz_v[sl]
        sxrc = sxr_v[sl]
        syrc = syr_v[sl]
        szrc = szr_v[sl]
        wtc = wt_v[sl]
        for half in range(L // S):
            bx = [jnp.full((L,), sxrc[half * S + u] * -2.0) for u in range(S)]
            by = [jnp.full((L,), syrc[half * S + u] * -2.0) for u in range(S)]
            bz = [jnp.full((L,), szrc[half * S + u] * -2.0) for u in range(S)]

            def tgt_body(ti, accs):
                accs = list(accs)
                tsl = pl.ds(ti * L, L)
                txc = txr_v[tsl]
                tyc = tyr_v[tsl]
                tzc = tzr_v[tsl]
                t2c = t2_v[tsl]
                for u in range(S):
                    v = t2c + bx[u] * txc
                    v = v + by[u] * tyc
                    v = v + bz[u] * tzc
                    accs[u] = jnp.minimum(accs[u], v)
                return tuple(accs)

            accs = lax.fori_loop(0, m_total // L, tgt_body, (inf,) * S)

            for u in range(S):
                i = half * S + u
                s2 = sxc[i] * sxc[i] + syc[i] * syc[i] + szc[i] * szc[i]
                total = total + (lane_min(accs[u]) + s2) * wtc[i]
        return total

    total = lax.fori_loop(0, n_per_w // L, chunk_body, jnp.float32(0.0))

    lane = lax.iota(jnp.int32, L)
    res_v[...] = jnp.where(lane == 0, jnp.full((L,), total), 0.0)
    pltpu.sync_copy(res_v, out.at[wid])


def _sc_chamfer(sT, sTr, tT, tTr, weights, ntc):
    B, _, N = sT.shape
    M = tT.shape[2]
    nsc = N - ntc
    n_per_w = (B * nsc) // NW
    assert B * nsc == n_per_w * NW and nsc % n_per_w == 0
    assert M % L == 0 and n_per_w % L == 0

    mesh = plsc.VectorSubcoreMesh(
        core_axis_name="c", subcore_axis_name="s",
        num_cores=NC, num_subcores=NS)
    body = functools.partial(_sc_body, n_per_w, nsc, M)
    partials = pl.kernel(
        body,
        out_type=jax.ShapeDtypeStruct((NW, L), jnp.float32),
        mesh=mesh,
        scratch_types=(
            [pltpu.VMEM((n_per_w,), jnp.float32)] * 7
            + [pltpu.VMEM((M,), jnp.float32)] * 7
            + [pltpu.VMEM((L,), jnp.float32)]
        ),
    )(
        sT[:, 0, ntc:].reshape(-1), sT[:, 1, ntc:].reshape(-1),
        sT[:, 2, ntc:].reshape(-1),
        sTr[:, 0, ntc:].reshape(-1), sTr[:, 1, ntc:].reshape(-1),
        sTr[:, 2, ntc:].reshape(-1),
        weights[:, ntc:].reshape(-1),
        tT[:, 0].reshape(-1), tT[:, 1].reshape(-1), tT[:, 2].reshape(-1),
        tTr[:, 0].reshape(-1), tTr[:, 1].reshape(-1), tTr[:, 2].reshape(-1),
    )
    return jnp.sum(partials)


def _tc_body(s_pre, t_bf, t_raw, s_raw, w, out, macc):
    mi = pl.program_id(1)
    nm = pl.num_programs(1)
    cross2 = lax.dot_general(
        s_pre[0], t_bf[0], (((0,), (0,)), ((), ())),
        preferred_element_type=jnp.float32)
    tr = t_raw[0]
    t2 = tr[0] * tr[0] + tr[1] * tr[1] + tr[2] * tr[2]
    part = jnp.min(cross2 + t2[None, :], axis=1)

    @pl.when(mi == 0)
    def _():
        macc[...] = part

    @pl.when(mi > 0)
    def _():
        macc[...] = jnp.minimum(macc[...], part)

    @pl.when(mi == nm - 1)
    def _():
        sr = s_raw[0]
        s2 = sr[0] * sr[0] + sr[1] * sr[1] + sr[2] * sr[2]
        val = jnp.sum((macc[...] + s2) * w[0, 0])
        row = lax.broadcasted_iota(jnp.int32, (8, 128), 0)
        col = lax.broadcasted_iota(jnp.int32, (8, 128), 1)
        mask = (row == 0) & (col == 0)
        out[...] = jnp.where(mask, val, 0.0)[None]


def _tc_chamfer(sT, s_pre, tT, t_bf, w3, ntc):
    B, _, N = sT.shape
    M = tT.shape[2]
    assert M % TM == 0
    out = pl.pallas_call(
        _tc_body,
        grid=(B, M // TM),
        in_specs=[
            pl.BlockSpec((1, 3, ntc), lambda b, m: (b, 0, 0)),
            pl.BlockSpec((1, 3, TM), lambda b, m: (b, 0, m)),
            pl.BlockSpec((1, 3, TM), lambda b, m: (b, 0, m)),
            pl.BlockSpec((1, 3, ntc), lambda b, m: (b, 0, 0)),
            pl.BlockSpec((1, 1, ntc), lambda b, m: (b, 0, 0)),
        ],
        out_specs=pl.BlockSpec((1, 8, 128), lambda b, m: (b, 0, 0)),
        out_shape=jax.ShapeDtypeStruct((B, 8, 128), jnp.float32),
        scratch_shapes=[pltpu.VMEM((ntc,), jnp.float32)],
    )(s_pre, t_bf, tT, sT, w3)
    return jnp.sum(out)


def kernel(source_cloud, target_cloud, weights_source):
    B, N, D = source_cloud.shape
    M = target_cloud.shape[1]
    ntc = N - NSC

    def round_bf16(x):
        u = lax.bitcast_convert_type(x, jnp.uint32)
        r = (u + jnp.uint32(0x7FFF) + ((u >> 16) & jnp.uint32(1)))
        r = r & jnp.uint32(0xFFFF0000)
        return lax.bitcast_convert_type(r, jnp.float32)

    sT = jnp.transpose(source_cloud, (0, 2, 1))
    tT = jnp.transpose(target_cloud, (0, 2, 1))
    sTr = round_bf16(sT)
    tTr = round_bf16(tT)

    s_pre = (-2.0 * sTr).astype(jnp.bfloat16)
    t_bf = tTr.astype(jnp.bfloat16)
    w3 = weights_source[:, None, :]

    tc_sum = _tc_chamfer(sT, s_pre, tT, t_bf, w3, ntc)
    sc_sum = _sc_chamfer(sT, sTr, tT, tTr, weights_source, ntc)

    return (tc_sum + sc_sum) / B


# speedup vs baseline: 1.0468x; 1.0353x over previous
import functools

import jax
import jax.numpy as jnp
from jax import lax
from jax.experimental import pallas as pl
from jax.experimental.pallas import tpu as pltpu
from jax.experimental.pallas import tpu_sc as plsc

NC = 2
NS = 16
NW = NC * NS
L = 16
S = 8
NSC = 256
TM = 2048


def _sc_body(n_per_w, n_total, m_total,
             sx, sy, sz, sxr, syr, szr, wt,
             tx, ty, tz, txr, tyr, tzr, out,
             sx_v, sy_v, sz_v, sxr_v, syr_v, szr_v, wt_v,
             tx_v, ty_v, tz_v, txr_v, tyr_v, tzr_v, t2_v, res_v):
    cid = lax.axis_index("c")
    sid = lax.axis_index("s")
    wid = sid * NC + cid
    base = wid * n_per_w
    tbase = (base // n_total) * m_total

    pltpu.sync_copy(sx.at[pl.ds(base, n_per_w)], sx_v)
    pltpu.sync_copy(sy.at[pl.ds(base, n_per_w)], sy_v)
    pltpu.sync_copy(sz.at[pl.ds(base, n_per_w)], sz_v)
    pltpu.sync_copy(sxr.at[pl.ds(base, n_per_w)], sxr_v)
    pltpu.sync_copy(syr.at[pl.ds(base, n_per_w)], syr_v)
    pltpu.sync_copy(szr.at[pl.ds(base, n_per_w)], szr_v)
    pltpu.sync_copy(wt.at[pl.ds(base, n_per_w)], wt_v)
    pltpu.sync_copy(tx.at[pl.ds(tbase, m_total)], tx_v)
    pltpu.sync_copy(ty.at[pl.ds(tbase, m_total)], ty_v)
    pltpu.sync_copy(tz.at[pl.ds(tbase, m_total)], tz_v)
    pltpu.sync_copy(txr.at[pl.ds(tbase, m_total)], txr_v)
    pltpu.sync_copy(tyr.at[pl.ds(tbase, m_total)], tyr_v)
    pltpu.sync_copy(tzr.at[pl.ds(tbase, m_total)], tzr_v)

    def prep(j, carry):
        sl = pl.ds(j * L, L)
        a = tx_v[sl]
        bb = ty_v[sl]
        c = tz_v[sl]
        t2_v[sl] = a * a + bb * bb + c * c
        return carry
    lax.fori_loop(0, m_total // L, prep, 0)

    inf = jnp.full((L,), jnp.float32(jnp.inf))

    dnums = lax.GatherDimensionNumbers(
        offset_dims=(), collapsed_slice_dims=(0,), start_index_map=(0,))

    def lane_rot(v, sh):
        perm = (lax.iota(jnp.int32, L) + sh) % L
        return lax.gather(
            v, perm[:, None], dnums, slice_sizes=(1,),
            mode=lax.GatherScatterMode.PROMISE_IN_BOUNDS)

    def lane_min(v):
        for sh in (8, 4, 2, 1):
            v = jnp.minimum(v, lane_rot(v, sh))
        return v[0]

    def chunk_body(ci, total):
        sl = pl.ds(ci * L, L)
        sxc = sx_v[sl]
        syc = sy_v[sl]
        szc = sz_v[sl]
        sxrc = sxr_v[sl]
        syrc = syr_v[sl]
        szrc = szr_v[sl]
        wtc = wt_v[sl]
        for half in range(L // S):
            bx = [jnp.full((L,), sxrc[half * S + u] * -2.0) for u in range(S)]
            by = [jnp.full((L,), syrc[half * S + u] * -2.0) for u in range(S)]
            bz = [jnp.full((L,), szrc[half * S + u] * -2.0) for u in range(S)]

            def tgt_body(ti, accs):
                accs = list(accs)
                tsl = pl.ds(ti * L, L)
                txc = txr_v[tsl]
                tyc = tyr_v[tsl]
                tzc = tzr_v[tsl]
                t2c = t2_v[tsl]
                for u in range(S):
                    v = t2c + bx[u] * txc
                    v = v + by[u] * tyc
                    v = v + bz[u] * tzc
                    accs[u] = jnp.minimum(accs[u], v)
                return tuple(accs)

            accs = lax.fori_loop(0, m_total // L, tgt_body, (inf,) * S)

            for u in range(S):
                i = half * S + u
                s2 = sxc[i] * sxc[i] + syc[i] * syc[i] + szc[i] * szc[i]
                total = total + (lane_min(accs[u]) + s2) * wtc[i]
        return total

    total = lax.fori_loop(0, n_per_w // L, chunk_body, jnp.float32(0.0))

    lane = lax.iota(jnp.int32, L)
    res_v[...] = jnp.where(lane == 0, jnp.full((L,), total), 0.0)
    pltpu.sync_copy(res_v, out.at[wid])


def _sc_chamfer(sT, sTr, tT, tTr, weights, ntc):
    B, _, N = sT.shape
    M = tT.shape[2]
    nsc = N - ntc
    n_per_w = (B * nsc) // NW
    assert B * nsc == n_per_w * NW and nsc % n_per_w == 0
    assert M % L == 0 and n_per_w % L == 0

    mesh = plsc.VectorSubcoreMesh(
        core_axis_name="c", subcore_axis_name="s",
        num_cores=NC, num_subcores=NS)
    body = functools.partial(_sc_body, n_per_w, nsc, M)
    partials = pl.kernel(
        body,
        out_type=jax.ShapeDtypeStruct((NW, L), jnp.float32),
        mesh=mesh,
        scratch_types=(
            [pltpu.VMEM((n_per_w,), jnp.float32)] * 7
            + [pltpu.VMEM((M,), jnp.float32)] * 7
            + [pltpu.VMEM((L,), jnp.float32)]
        ),
    )(
        sT[:, 0, ntc:].reshape(-1), sT[:, 1, ntc:].reshape(-1),
        sT[:, 2, ntc:].reshape(-1),
        sTr[:, 0, ntc:].reshape(-1), sTr[:, 1, ntc:].reshape(-1),
        sTr[:, 2, ntc:].reshape(-1),
        weights[:, ntc:].reshape(-1),
        tT[:, 0].reshape(-1), tT[:, 1].reshape(-1), tT[:, 2].reshape(-1),
        tTr[:, 0].reshape(-1), tTr[:, 1].reshape(-1), tTr[:, 2].reshape(-1),
    )
    return jnp.sum(partials)


def _tc_body(s_pre, t_bf, t_raw, s_raw, w, out, macc):
    mi = pl.program_id(1)
    nm = pl.num_programs(1)
    cross2 = lax.dot_general(
        s_pre[0], t_bf[0], (((0,), (0,)), ((), ())),
        preferred_element_type=jnp.float32)
    tr = t_raw[0]
    t2 = tr[0] * tr[0] + tr[1] * tr[1] + tr[2] * tr[2]
    part = jnp.min(cross2 + t2[None, :], axis=1)

    @pl.when(mi == 0)
    def _():
        macc[...] = part

    @pl.when(mi > 0)
    def _():
        macc[...] = jnp.minimum(macc[...], part)

    @pl.when(mi == nm - 1)
    def _():
        sr = s_raw[0]
        s2 = sr[0] * sr[0] + sr[1] * sr[1] + sr[2] * sr[2]
        val = jnp.sum((macc[...] + s2) * w[0, 0])
        row = lax.broadcasted_iota(jnp.int32, (8, 128), 0)
        col = lax.broadcasted_iota(jnp.int32, (8, 128), 1)
        mask = (row == 0) & (col == 0)
        out[...] = jnp.where(mask, val, 0.0)[None]


def _tc_chamfer(sT, s_pre, tT, t_bf, w3, ntc):
    B, _, N = sT.shape
    M = tT.shape[2]
    assert M % TM == 0
    out = pl.pallas_call(
        _tc_body,
        grid=(B, M // TM),
        in_specs=[
            pl.BlockSpec((1, 3, ntc), lambda b, m: (b, 0, 0)),
            pl.BlockSpec((1, 3, TM), lambda b, m: (b, 0, m)),
            pl.BlockSpec((1, 3, TM), lambda b, m: (b, 0, m)),
            pl.BlockSpec((1, 3, ntc), lambda b, m: (b, 0, 0)),
            pl.BlockSpec((1, 1, ntc), lambda b, m: (b, 0, 0)),
        ],
        out_specs=pl.BlockSpec((1, 8, 128), lambda b, m: (b, 0, 0)),
        out_shape=jax.ShapeDtypeStruct((B, 8, 128), jnp.float32),
        scratch_shapes=[pltpu.VMEM((ntc,), jnp.float32)],
    )(s_pre, t_bf, tT, sT, w3)
    return jnp.sum(out)


def kernel(source_cloud, target_cloud, weights_source):
    B, N, D = source_cloud.shape
    M = target_cloud.shape[1]
    ntc = N - NSC

    def round_bf16(x):
        u = lax.bitcast_convert_type(x, jnp.uint32)
        r = (u + jnp.uint32(0x7FFF) + ((u >> 16) & jnp.uint32(1)))
        r = r & jnp.uint32(0xFFFF0000)
        return lax.bitcast_convert_type(r, jnp.float32)

    sT = jnp.transpose(source_cloud, (0, 2, 1))
    tT = jnp.transpose(target_cloud, (0, 2, 1))
    sTr = round_bf16(sT)
    tTr = round_bf16(tT)

    s_pre = (-2.0 * sTr).astype(jnp.bfloat16)
    t_bf = tTr.astype(jnp.bfloat16)
    w3 = weights_source[:, None, :]

    tc_sum = _tc_chamfer(sT, s_pre, tT, t_bf, w3, ntc)
    sc_sum = _sc_chamfer(sT, sTr, tT, tTr, weights_source, ntc)

    return (tc_sum + sc_sum) / B


# speedup vs baseline: 1.0919x; 1.0431x over previous
import functools

import jax
import jax.numpy as jnp
from jax import lax
from jax.experimental import pallas as pl
from jax.experimental.pallas import tpu as pltpu
from jax.experimental.pallas import tpu_sc as plsc

NC = 2
NS = 16
NW = NC * NS
L = 16
S = 8
NSC = 256
TM = 2048


def _sc_body(n_per_w, n_total, m_total,
             sx, sy, sz, sxr, syr, szr, wt,
             tx, ty, tz, txr, tyr, tzr, out,
             sx_v, sy_v, sz_v, sxr_v, syr_v, szr_v, wt_v,
             tx_v, ty_v, tz_v, txr_v, tyr_v, tzr_v, t2_v, res_v):
    cid = lax.axis_index("c")
    sid = lax.axis_index("s")
    wid = sid * NC + cid
    base = wid * n_per_w
    tbase = (base // n_total) * m_total

    pltpu.sync_copy(sx.at[pl.ds(base, n_per_w)], sx_v)
    pltpu.sync_copy(sy.at[pl.ds(base, n_per_w)], sy_v)
    pltpu.sync_copy(sz.at[pl.ds(base, n_per_w)], sz_v)
    pltpu.sync_copy(sxr.at[pl.ds(base, n_per_w)], sxr_v)
    pltpu.sync_copy(syr.at[pl.ds(base, n_per_w)], syr_v)
    pltpu.sync_copy(szr.at[pl.ds(base, n_per_w)], szr_v)
    pltpu.sync_copy(wt.at[pl.ds(base, n_per_w)], wt_v)
    pltpu.sync_copy(tx.at[pl.ds(tbase, m_total)], tx_v)
    pltpu.sync_copy(ty.at[pl.ds(tbase, m_total)], ty_v)
    pltpu.sync_copy(tz.at[pl.ds(tbase, m_total)], tz_v)
    pltpu.sync_copy(txr.at[pl.ds(tbase, m_total)], txr_v)
    pltpu.sync_copy(tyr.at[pl.ds(tbase, m_total)], tyr_v)
    pltpu.sync_copy(tzr.at[pl.ds(tbase, m_total)], tzr_v)

    def prep(j, carry):
        sl = pl.ds(j * L, L)
        a = tx_v[sl]
        bb = ty_v[sl]
        c = tz_v[sl]
        t2_v[sl] = a * a + bb * bb + c * c
        return carry
    lax.fori_loop(0, m_total // L, prep, 0)

    inf = jnp.full((L,), jnp.float32(jnp.inf))

    dnums = lax.GatherDimensionNumbers(
        offset_dims=(), collapsed_slice_dims=(0,), start_index_map=(0,))

    def lane_rot(v, sh):
        perm = (lax.iota(jnp.int32, L) + sh) % L
        return lax.gather(
            v, perm[:, None], dnums, slice_sizes=(1,),
            mode=lax.GatherScatterMode.PROMISE_IN_BOUNDS)

    def lane_min(v):
        for sh in (8, 4, 2, 1):
            v = jnp.minimum(v, lane_rot(v, sh))
        return v[0]

    def chunk_body(ci, total):
        sl = pl.ds(ci * L, L)
        sxc = sx_v[sl]
        syc = sy_v[sl]
        szc = sz_v[sl]
        sxrc = sxr_v[sl]
        syrc = syr_v[sl]
        szrc = szr_v[sl]
        wtc = wt_v[sl]
        for half in range(L // S):
            bx = [jnp.full((L,), sxrc[half * S + u] * -2.0) for u in range(S)]
            by = [jnp.full((L,), syrc[half * S + u] * -2.0) for u in range(S)]
            bz = [jnp.full((L,), szrc[half * S + u] * -2.0) for u in range(S)]

            def tgt_body(ti, accs):
                accs = list(accs)
                tsl = pl.ds(ti * L, L)
                txc = txr_v[tsl]
                tyc = tyr_v[tsl]
                tzc = tzr_v[tsl]
                t2c = t2_v[tsl]
                for u in range(S):
                    v = t2c + bx[u] * txc
                    v = v + by[u] * tyc
                    v = v + bz[u] * tzc
                    accs[u] = jnp.minimum(accs[u], v)
                return tuple(accs)

            accs = lax.fori_loop(0, m_total // L, tgt_body, (inf,) * S)

            for u in range(S):
                i = half * S + u
                s2 = sxc[i] * sxc[i] + syc[i] * syc[i] + szc[i] * szc[i]
                total = total + (lane_min(accs[u]) + s2) * wtc[i]
        return total

    total = lax.fori_loop(0, n_per_w // L, chunk_body, jnp.float32(0.0))

    lane = lax.iota(jnp.int32, L)
    res_v[...] = jnp.where(lane == 0, jnp.full((L,), total), 0.0)
    pltpu.sync_copy(res_v, out.at[wid])


def _sc_chamfer(sT, sTr, tT, tTr, weights, ntc):
    B, _, N = sT.shape
    M = tT.shape[2]
    nsc = N - ntc
    n_per_w = (B * nsc) // NW
    assert B * nsc == n_per_w * NW and nsc % n_per_w == 0
    assert M % L == 0 and n_per_w % L == 0

    mesh = plsc.VectorSubcoreMesh(
        core_axis_name="c", subcore_axis_name="s",
        num_cores=NC, num_subcores=NS)
    body = functools.partial(_sc_body, n_per_w, nsc, M)
    partials = pl.kernel(
        body,
        out_type=jax.ShapeDtypeStruct((NW, L), jnp.float32),
        mesh=mesh,
        scratch_types=(
            [pltpu.VMEM((n_per_w,), jnp.float32)] * 7
            + [pltpu.VMEM((M,), jnp.float32)] * 7
            + [pltpu.VMEM((L,), jnp.float32)]
        ),
    )(
        sT[:, 0, ntc:].reshape(-1), sT[:, 1, ntc:].reshape(-1),
        sT[:, 2, ntc:].reshape(-1),
        sTr[:, 0, ntc:].reshape(-1), sTr[:, 1, ntc:].reshape(-1),
        sTr[:, 2, ntc:].reshape(-1),
        weights[:, ntc:].reshape(-1),
        tT[:, 0].reshape(-1), tT[:, 1].reshape(-1), tT[:, 2].reshape(-1),
        tTr[:, 0].reshape(-1), tTr[:, 1].reshape(-1), tTr[:, 2].reshape(-1),
    )
    return jnp.sum(partials)


def _tc_body(s_pre, t_bf, t_raw, s_raw, w, out, macc):
    mi = pl.program_id(1)
    nm = pl.num_programs(1)
    cross2 = lax.dot_general(
        s_pre[0], t_bf[0], (((0,), (0,)), ((), ())),
        preferred_element_type=jnp.float32)
    tr = t_raw[0]
    t2 = tr[0] * tr[0] + tr[1] * tr[1] + tr[2] * tr[2]
    cand = cross2 + t2[None, :]
    tm = cand.shape[1]
    part = cand[:, 0:128]
    for k in range(1, tm // 128):
        part = jnp.minimum(part, cand[:, k * 128:(k + 1) * 128])

    @pl.when(mi == 0)
    def _():
        macc[...] = part

    @pl.when(mi > 0)
    def _():
        macc[...] = jnp.minimum(macc[...], part)

    @pl.when(mi == nm - 1)
    def _():
        sr = s_raw[0]
        s2 = sr[0] * sr[0] + sr[1] * sr[1] + sr[2] * sr[2]
        dmin = jnp.min(macc[...], axis=1)
        val = jnp.sum((dmin + s2) * w[0, 0])
        row = lax.broadcasted_iota(jnp.int32, (8, 128), 0)
        col = lax.broadcasted_iota(jnp.int32, (8, 128), 1)
        mask = (row == 0) & (col == 0)
        out[...] = jnp.where(mask, val, 0.0)[None]


def _tc_chamfer(sT, s_pre, tT, t_bf, w3, ntc):
    B, _, N = sT.shape
    M = tT.shape[2]
    assert M % TM == 0
    out = pl.pallas_call(
        _tc_body,
        grid=(B, M // TM),
        in_specs=[
            pl.BlockSpec((1, 3, ntc), lambda b, m: (b, 0, 0)),
            pl.BlockSpec((1, 3, TM), lambda b, m: (b, 0, m)),
            pl.BlockSpec((1, 3, TM), lambda b, m: (b, 0, m)),
            pl.BlockSpec((1, 3, ntc), lambda b, m: (b, 0, 0)),
            pl.BlockSpec((1, 1, ntc), lambda b, m: (b, 0, 0)),
        ],
        out_specs=pl.BlockSpec((1, 8, 128), lambda b, m: (b, 0, 0)),
        out_shape=jax.ShapeDtypeStruct((B, 8, 128), jnp.float32),
        scratch_shapes=[pltpu.VMEM((ntc, 128), jnp.float32)],
    )(s_pre, t_bf, tT, sT, w3)
    return jnp.sum(out)


def kernel(source_cloud, target_cloud, weights_source):
    B, N, D = source_cloud.shape
    M = target_cloud.shape[1]
    ntc = N - NSC

    def round_bf16(x):
        u = lax.bitcast_convert_type(x, jnp.uint32)
        r = (u + jnp.uint32(0x7FFF) + ((u >> 16) & jnp.uint32(1)))
        r = r & jnp.uint32(0xFFFF0000)
        return lax.bitcast_convert_type(r, jnp.float32)

    sT = jnp.transpose(source_cloud, (0, 2, 1))
    tT = jnp.transpose(target_cloud, (0, 2, 1))
    sTr = round_bf16(sT)
    tTr = round_bf16(tT)

    s_pre = (-2.0 * sTr).astype(jnp.bfloat16)
    t_bf = tTr.astype(jnp.bfloat16)
    w3 = weights_source[:, None, :]

    tc_sum = _tc_chamfer(sT, s_pre, tT, t_bf, w3, ntc)
    sc_sum = _sc_chamfer(sT, sTr, tT, tTr, weights_source, ntc)

    return (tc_sum + sc_sum) / B
